# baseline (speedup 1.0000x reference)
import functools

import jax
import jax.numpy as jnp
from jax import lax
from jax.experimental import pallas as pl
from jax.experimental.pallas import tpu as pltpu
from jax.experimental.pallas import tpu_sc as plsc

_K = 16
_N = 512
_B = 16
_D = 128
_L = 16
_NCHUNK = _N // _L
_NW = 32
_RPT = _B * _N // _NW


def _sc_topk_body(x_hbm, y_hbm, out_hbm, xv, yv, outbuf):
    cid = lax.axis_index("c")
    sid = lax.axis_index("s")
    wid = sid * 2 + cid
    batch = wid // 2
    half = wid % 2
    pltpu.sync_copy(x_hbm.at[batch], xv)
    pltpu.sync_copy(y_hbm.at[batch], yv)

    def row_body(r, carry):
        i = half * _RPT + r
        iv = jnp.zeros((_L,), jnp.int32) + i
        xi = plsc.load_gather(xv, [iv])
        yi = plsc.load_gather(yv, [iv])
        ak = av = None
        for c in range(_NCHUNK):
            cols = lax.iota(jnp.int32, _L) + c * _L
            dx = xv[pl.ds(c * _L, _L)] - xi
            dy = yv[pl.ds(c * _L, _L)] - yi
            d2 = dx * dx + dy * dy
            d2 = jnp.where(cols == i, jnp.float32(jnp.inf), d2)
            if c == 0:
                ak, av = plsc.sort_key_val(d2, cols)
            else:
                bk, bv = plsc.sort_key_val(d2, cols, descending=True)
                take = bk < ak
                lk = jnp.where(take, bk, ak)
                lv = jnp.where(take, bv, av)
                ak, av = plsc.sort_key_val(lk, lv)
        gx = plsc.load_gather(xv, [av])
        gy = plsc.load_gather(yv, [av])
        outbuf[r, pl.ds(0, _L)] = gx - xi
        outbuf[r, pl.ds(_L, _L)] = gy - yi
        return carry

    lax.fori_loop(0, _RPT, row_body, 0)
    pltpu.sync_copy(outbuf, out_hbm.at[pl.ds(wid * _RPT, _RPT)])


def _mm_body(locs_ref, rel_ref, W2_ref, Wr_ref, b_ref, out_ref):
    out_ref[...] = (
        jnp.dot(rel_ref[...], Wr_ref[...], preferred_element_type=jnp.float32)
        + jnp.dot(locs_ref[...], W2_ref[...], preferred_element_type=jnp.float32)
        + b_ref[...]
    )


@jax.jit
def kernel(locs, W, b):
    B, N, _ = locs.shape
    x = locs[:, :, 0] + 0.0
    y = locs[:, :, 1] + 0.0
    mesh = plsc.VectorSubcoreMesh(core_axis_name="c", subcore_axis_name="s")
    rel = pl.kernel(
        _sc_topk_body,
        out_type=jax.ShapeDtypeStruct((B * N, 2 * _L), jnp.float32),
        mesh=mesh,
        compiler_params=pltpu.CompilerParams(needs_layout_passes=False),
        scratch_types=[
            pltpu.VMEM((N,), jnp.float32),
            pltpu.VMEM((N,), jnp.float32),
            pltpu.VMEM((_RPT, 2 * _L), jnp.float32),
        ],
    )(x, y)

    order = [2 + 2 * k for k in range(_K)] + [3 + 2 * k for k in range(_K)]
    Wr = W[jnp.asarray(order)]
    W2 = W[:2]
    b2 = b.reshape(1, _D)
    locs_flat = locs.reshape(B * N, 2)
    MB = 1024
    out = pl.pallas_call(
        _mm_body,
        grid=(B * N // MB,),
        in_specs=[
            pl.BlockSpec((MB, 2), lambda i: (i, 0)),
            pl.BlockSpec((MB, 2 * _L), lambda i: (i, 0)),
            pl.BlockSpec((2, _D), lambda i: (0, 0)),
            pl.BlockSpec((2 * _L, _D), lambda i: (0, 0)),
            pl.BlockSpec((1, _D), lambda i: (0, 0)),
        ],
        out_specs=pl.BlockSpec((MB, _D), lambda i: (i, 0)),
        out_shape=jax.ShapeDtypeStruct((B * N, _D), jnp.float32),
    )(locs_flat, rel, W2, Wr, b2)
    return out.reshape(B, N, _D)


# speedup vs baseline: 1.0034x; 1.0034x over previous
import functools

import jax
import jax.numpy as jnp
from jax import lax
from jax.experimental import pallas as pl
from jax.experimental.pallas import tpu as pltpu
from jax.experimental.pallas import tpu_sc as plsc

_K = 16
_N = 512
_B = 16
_D = 128
_L = 16
_NCHUNK = _N // _L
_NW = 32
_RPT = _B * _N // _NW
_F = 40


def _sc_topk_body(locs_hbm, out_hbm, locv, outbuf):
    cid = lax.axis_index("c")
    sid = lax.axis_index("s")
    wid = sid * 2 + cid
    batch = wid // 2
    half = wid % 2
    pltpu.sync_copy(locs_hbm.at[batch], locv)

    lane = lax.iota(jnp.int32, _L)
    lane2 = lane * 2

    def row_body(r, carry):
        i = half * _RPT + r
        iv2 = jnp.zeros((_L,), jnp.int32) + 2 * i
        xi = plsc.load_gather(locv, [iv2])
        yi = plsc.load_gather(locv, [iv2 + 1])
        ak = av = None
        for c in range(_NCHUNK):
            cols = lane + c * _L
            cols2 = lane2 + c * (2 * _L)
            dx = plsc.load_gather(locv, [cols2]) - xi
            dy = plsc.load_gather(locv, [cols2 + 1]) - yi
            d2 = dx * dx + dy * dy
            d2 = jnp.where(cols == i, jnp.float32(jnp.inf), d2)
            if c == 0:
                ak, av = plsc.sort_key_val(d2, cols)
            else:
                bk, bv = plsc.sort_key_val(d2, cols, descending=True)
                take = bk < ak
                lk = jnp.where(take, bk, ak)
                lv = jnp.where(take, bv, av)
                ak, av = plsc.sort_key_val(lk, lv)
        av2 = av * 2
        gx = plsc.load_gather(locv, [av2])
        gy = plsc.load_gather(locv, [av2 + 1])
        head = jnp.where(lane == 1, yi, xi)
        outbuf[r, pl.ds(0, _L)] = head
        rv = jnp.zeros((_L,), jnp.int32) + r
        plsc.store_scatter(outbuf, [rv, lane2 + 2], gx - xi)
        plsc.store_scatter(outbuf, [rv, lane2 + 3], gy - yi)
        return carry

    lax.fori_loop(0, _RPT, row_body, 0)
    pltpu.sync_copy(outbuf, out_hbm.at[pl.ds(wid * _RPT, _RPT)])


def _mm_body(feats_ref, W_ref, b_ref, out_ref):
    out_ref[...] = (
        jnp.dot(feats_ref[...], W_ref[...], preferred_element_type=jnp.float32)
        + b_ref[...]
    )


@jax.jit
def kernel(locs, W, b):
    B, N, _ = locs.shape
    locs_flat = locs.reshape(B, 2 * N)
    mesh = plsc.VectorSubcoreMesh(core_axis_name="c", subcore_axis_name="s")
    feats = pl.kernel(
        _sc_topk_body,
        out_type=jax.ShapeDtypeStruct((B * N, _F), jnp.float32),
        mesh=mesh,
        compiler_params=pltpu.CompilerParams(needs_layout_passes=False),
        scratch_types=[
            pltpu.VMEM((2 * N,), jnp.float32),
            pltpu.VMEM((_RPT, _F), jnp.float32),
        ],
    )(locs_flat)

    Wpad = jnp.zeros((_F, _D), W.dtype).at[:34].set(W)
    b2 = b.reshape(1, _D)
    MB = 1024
    out = pl.pallas_call(
        _mm_body,
        grid=(B * N // MB,),
        in_specs=[
            pl.BlockSpec((MB, _F), lambda i: (i, 0)),
            pl.BlockSpec((_F, _D), lambda i: (0, 0)),
            pl.BlockSpec((1, _D), lambda i: (0, 0)),
        ],
        out_specs=pl.BlockSpec((MB, _D), lambda i: (i, 0)),
        out_shape=jax.ShapeDtypeStruct((B * N, _D), jnp.float32),
    )(feats, Wpad, b2)
    return out.reshape(B, N, _D)


# speedup vs baseline: 1.5995x; 1.5940x over previous
import functools

import jax
import jax.numpy as jnp
from jax import lax
from jax.experimental import pallas as pl
from jax.experimental.pallas import tpu as pltpu
from jax.experimental.pallas import tpu_sc as plsc

_K = 16
_N = 512
_B = 16
_D = 128
_L = 16
_NCHUNK = _N // _L
_NW = 32
_F = 40
_BTC = 8
_BSC = _B - _BTC
_TPB = _NW // _BSC
_RPT = _N // _TPB


def _sc_topk_body(locs_hbm, out_hbm, locv, outbuf):
    cid = lax.axis_index("c")
    sid = lax.axis_index("s")
    wid = sid * 2 + cid
    batch = _BTC + wid // _TPB
    part = wid % _TPB
    pltpu.sync_copy(locs_hbm.at[batch], locv)

    lane = lax.iota(jnp.int32, _L)
    lane2 = lane * 2

    def row_body(r, carry):
        i = part * _RPT + r
        iv2 = jnp.zeros((_L,), jnp.int32) + 2 * i
        xi = plsc.load_gather(locv, [iv2])
        yi = plsc.load_gather(locv, [iv2 + 1])
        ak = av = None
        for c in range(_NCHUNK):
            cols = lane + c * _L
            cols2 = lane2 + c * (2 * _L)
            dx = plsc.load_gather(locv, [cols2]) - xi
            dy = plsc.load_gather(locv, [cols2 + 1]) - yi
            d2 = dx * dx + dy * dy
            d2 = jnp.where(cols == i, jnp.float32(jnp.inf), d2)
            if c == 0:
                ak, av = plsc.sort_key_val(d2, cols)
            else:
                bk, bv = plsc.sort_key_val(d2, cols, descending=True)
                take = bk < ak
                lk = jnp.where(take, bk, ak)
                lv = jnp.where(take, bv, av)
                ak, av = plsc.sort_key_val(lk, lv)
        av2 = av * 2
        gx = plsc.load_gather(locv, [av2])
        gy = plsc.load_gather(locv, [av2 + 1])
        head = jnp.where(lane == 1, yi, xi)
        outbuf[r, pl.ds(0, _L)] = head
        rv = jnp.zeros((_L,), jnp.int32) + r
        plsc.store_scatter(outbuf, [rv, lane2 + 2], gx - xi)
        plsc.store_scatter(outbuf, [rv, lane2 + 3], gy - yi)
        return carry

    lax.fori_loop(0, _RPT, row_body, 0)
    pltpu.sync_copy(outbuf, out_hbm.at[pl.ds(wid * _RPT, _RPT)])


def _mm_body(feats_ref, W_ref, b_ref, out_ref):
    out_ref[...] = (
        jnp.dot(feats_ref[...], W_ref[...], preferred_element_type=jnp.float32)
        + b_ref[...]
    )


def _tc_body(locsT_ref, locs_ref, Wp_ref, b_ref, out_ref, feats_ref):
    n = _N
    x_row = locsT_ref[0, 0:1, :]
    y_row = locsT_ref[0, 1:2, :]
    x_col = locs_ref[0, :, 0:1]
    y_col = locs_ref[0, :, 1:2]
    dxm = x_row - x_col
    dym = y_row - y_col
    d2 = dxm * dxm + dym * dym
    colj = jax.lax.broadcasted_iota(jnp.int32, (n, n), 1)
    rowi = jax.lax.broadcasted_iota(jnp.int32, (n, n), 0)
    inf = jnp.float32(jnp.inf)
    d2 = jnp.where(rowi == colj, inf, d2)

    feats_ref[:, 0:2] = locs_ref[0]
    feats_ref[:, 34:_F] = jnp.zeros((n, _F - 34), jnp.float32)
    for k in range(_K):
        mind2 = jnp.min(d2, axis=1, keepdims=True)
        mask = d2 == mind2
        feats_ref[:, 2 + k:3 + k] = jnp.sum(
            jnp.where(mask, dxm, 0.0), axis=1, keepdims=True)
        feats_ref[:, 18 + k:19 + k] = jnp.sum(
            jnp.where(mask, dym, 0.0), axis=1, keepdims=True)
        d2 = jnp.where(mask, inf, d2)

    out_ref[0] = (
        jnp.dot(feats_ref[...], Wp_ref[...], preferred_element_type=jnp.float32)
        + b_ref[...]
    )


@jax.jit
def kernel(locs, W, b):
    B, N, _ = locs.shape
    locs_flat = locs.reshape(B, 2 * N)
    b2 = b.reshape(1, _D)

    mesh = plsc.VectorSubcoreMesh(core_axis_name="c", subcore_axis_name="s")
    feats = pl.kernel(
        _sc_topk_body,
        out_type=jax.ShapeDtypeStruct((_BSC * N, _F), jnp.float32),
        mesh=mesh,
        compiler_params=pltpu.CompilerParams(needs_layout_passes=False),
        scratch_types=[
            pltpu.VMEM((2 * N,), jnp.float32),
            pltpu.VMEM((_RPT, _F), jnp.float32),
        ],
    )(locs_flat)

    locsT = locs.transpose(0, 2, 1)
    order = [0, 1] + [2 + 2 * k for k in range(_K)] + [3 + 2 * k for k in range(_K)]
    Wp = jnp.zeros((_F, _D), W.dtype).at[:34].set(W[jnp.asarray(order)])
    out_tc = pl.pallas_call(
        _tc_body,
        grid=(_BTC,),
        in_specs=[
            pl.BlockSpec((1, 2, N), lambda i: (i, 0, 0)),
            pl.BlockSpec((1, N, 2), lambda i: (i, 0, 0)),
            pl.BlockSpec((_F, _D), lambda i: (0, 0)),
            pl.BlockSpec((1, _D), lambda i: (0, 0)),
        ],
        out_specs=pl.BlockSpec((1, N, _D), lambda i: (i, 0, 0)),
        out_shape=jax.ShapeDtypeStruct((_BTC, N, _D), jnp.float32),
        scratch_shapes=[pltpu.VMEM((N, _F), jnp.float32)],
    )(locsT[:_BTC], locs[:_BTC], Wp, b2)

    Wpad = jnp.zeros((_F, _D), W.dtype).at[:34].set(W)
    MB = 1024
    out_sc = pl.pallas_call(
        _mm_body,
        grid=(_BSC * N // MB,),
        in_specs=[
            pl.BlockSpec((MB, _F), lambda i: (i, 0)),
            pl.BlockSpec((_F, _D), lambda i: (0, 0)),
            pl.BlockSpec((1, _D), lambda i: (0, 0)),
        ],
        out_specs=pl.BlockSpec((MB, _D), lambda i: (i, 0)),
        out_shape=jax.ShapeDtypeStruct((_BSC * N, _D), jnp.float32),
    )(feats, Wpad, b2)

    return jnp.concatenate([out_tc, out_sc.reshape(_BSC, N, _D)], axis=0)


# speedup vs baseline: 1.6121x; 1.0079x over previous
import functools

import jax
import jax.numpy as jnp
from jax import lax
from jax.experimental import pallas as pl
from jax.experimental.pallas import tpu as pltpu
from jax.experimental.pallas import tpu_sc as plsc

_K = 16
_N = 512
_B = 16
_D = 128
_L = 16
_NCHUNK = _N // _L
_NW = 32
_F = 40
_BTC = 8
_BSC = _B - _BTC
_TPB = _NW // _BSC
_RPT = _N // _TPB


def _sc_topk_body(locs_hbm, out_hbm, locv, outbuf):
    cid = lax.axis_index("c")
    sid = lax.axis_index("s")
    wid = sid * 2 + cid
    batch = _BTC + wid // _TPB
    part = wid % _TPB
    pltpu.sync_copy(locs_hbm.at[batch], locv)

    lane = lax.iota(jnp.int32, _L)
    lane2 = lane * 2
    hrpt = _RPT // 2

    def row_body(r, carry):
        iA = part * _RPT + r
        iB = iA + hrpt
        ivA = jnp.zeros((_L,), jnp.int32) + 2 * iA
        ivB = jnp.zeros((_L,), jnp.int32) + 2 * iB
        xiA = plsc.load_gather(locv, [ivA])
        yiA = plsc.load_gather(locv, [ivA + 1])
        xiB = plsc.load_gather(locv, [ivB])
        yiB = plsc.load_gather(locv, [ivB + 1])
        akA = avA = akB = avB = None
        for c in range(_NCHUNK):
            cols = lane + c * _L
            cols2 = lane2 + c * (2 * _L)
            cx = plsc.load_gather(locv, [cols2])
            cy = plsc.load_gather(locv, [cols2 + 1])
            dxA = cx - xiA
            dyA = cy - yiA
            dxB = cx - xiB
            dyB = cy - yiB
            d2A = dxA * dxA + dyA * dyA
            d2B = dxB * dxB + dyB * dyB
            d2A = jnp.where(cols == iA, jnp.float32(jnp.inf), d2A)
            d2B = jnp.where(cols == iB, jnp.float32(jnp.inf), d2B)
            if c == 0:
                akA, avA = plsc.sort_key_val(d2A, cols)
                akB, avB = plsc.sort_key_val(d2B, cols)
            else:
                bkA, bvA = plsc.sort_key_val(d2A, cols, descending=True)
                bkB, bvB = plsc.sort_key_val(d2B, cols, descending=True)
                tA = bkA < akA
                tB = bkB < akB
                lkA = jnp.where(tA, bkA, akA)
                lvA = jnp.where(tA, bvA, avA)
                lkB = jnp.where(tB, bkB, akB)
                lvB = jnp.where(tB, bvB, avB)
                akA, avA = plsc.sort_key_val(lkA, lvA)
                akB, avB = plsc.sort_key_val(lkB, lvB)
        for (i, xi, yi, av, roff) in ((iA, xiA, yiA, avA, 0),
                                      (iB, xiB, yiB, avB, hrpt)):
            av2 = av * 2
            gx = plsc.load_gather(locv, [av2])
            gy = plsc.load_gather(locv, [av2 + 1])
            head = jnp.where(lane == 1, yi, xi)
            outbuf[r + roff, pl.ds(0, _L)] = head
            rv = jnp.zeros((_L,), jnp.int32) + (r + roff)
            plsc.store_scatter(outbuf, [rv, lane2 + 2], gx - xi)
            plsc.store_scatter(outbuf, [rv, lane2 + 3], gy - yi)
        return carry

    lax.fori_loop(0, hrpt, row_body, 0)
    pltpu.sync_copy(outbuf, out_hbm.at[pl.ds(wid * _RPT, _RPT)])


def _mm_body(feats_ref, W_ref, b_ref, out_ref):
    out_ref[...] = (
        jnp.dot(feats_ref[...], W_ref[...], preferred_element_type=jnp.float32)
        + b_ref[...]
    )


def _tc_body(locsT_ref, locs_ref, Wp_ref, b_ref, out_ref, feats_ref):
    n = _N
    x_row = locsT_ref[0, 0:1, :]
    y_row = locsT_ref[0, 1:2, :]
    x_col = locs_ref[0, :, 0:1]
    y_col = locs_ref[0, :, 1:2]
    dxm = x_row - x_col
    dym = y_row - y_col
    d2 = dxm * dxm + dym * dym
    colj = jax.lax.broadcasted_iota(jnp.int32, (n, n), 1)
    rowi = jax.lax.broadcasted_iota(jnp.int32, (n, n), 0)
    inf = jnp.float32(jnp.inf)
    d2 = jnp.where(rowi == colj, inf, d2)

    feats_ref[:, 0:2] = locs_ref[0]
    feats_ref[:, 34:_F] = jnp.zeros((n, _F - 34), jnp.float32)
    for k in range(_K):
        mind2 = jnp.min(d2, axis=1, keepdims=True)
        mask = d2 == mind2
        feats_ref[:, 2 + k:3 + k] = jnp.sum(
            jnp.where(mask, dxm, 0.0), axis=1, keepdims=True)
        feats_ref[:, 18 + k:19 + k] = jnp.sum(
            jnp.where(mask, dym, 0.0), axis=1, keepdims=True)
        d2 = jnp.where(mask, inf, d2)

    out_ref[0] = (
        jnp.dot(feats_ref[...], Wp_ref[...], preferred_element_type=jnp.float32)
        + b_ref[...]
    )


@jax.jit
def kernel(locs, W, b):
    B, N, _ = locs.shape
    locs_flat = locs.reshape(B, 2 * N)
    b2 = b.reshape(1, _D)

    mesh = plsc.VectorSubcoreMesh(core_axis_name="c", subcore_axis_name="s")
    feats = pl.kernel(
        _sc_topk_body,
        out_type=jax.ShapeDtypeStruct((_BSC * N, _F), jnp.float32),
        mesh=mesh,
        compiler_params=pltpu.CompilerParams(needs_layout_passes=False),
        scratch_types=[
            pltpu.VMEM((2 * N,), jnp.float32),
            pltpu.VMEM((_RPT, _F), jnp.float32),
        ],
    )(locs_flat)

    locsT = locs.transpose(0, 2, 1)
    order = [0, 1] + [2 + 2 * k for k in range(_K)] + [3 + 2 * k for k in range(_K)]
    Wp = jnp.zeros((_F, _D), W.dtype).at[:34].set(W[jnp.asarray(order)])
    out_tc = pl.pallas_call(
        _tc_body,
        grid=(_BTC,),
        in_specs=[
            pl.BlockSpec((1, 2, N), lambda i: (i, 0, 0)),
            pl.BlockSpec((1, N, 2), lambda i: (i, 0, 0)),
            pl.BlockSpec((_F, _D), lambda i: (0, 0)),
            pl.BlockSpec((1, _D), lambda i: (0, 0)),
        ],
        out_specs=pl.BlockSpec((1, N, _D), lambda i: (i, 0, 0)),
        out_shape=jax.ShapeDtypeStruct((_BTC, N, _D), jnp.float32),
        scratch_shapes=[pltpu.VMEM((N, _F), jnp.float32)],
    )(locsT[:_BTC], locs[:_BTC], Wp, b2)

    Wpad = jnp.zeros((_F, _D), W.dtype).at[:34].set(W)
    MB = 1024
    out_sc = pl.pallas_call(
        _mm_body,
        grid=(_BSC * N // MB,),
        in_specs=[
            pl.BlockSpec((MB, _F), lambda i: (i, 0)),
            pl.BlockSpec((_F, _D), lambda i: (0, 0)),
            pl.BlockSpec((1, _D), lambda i: (0, 0)),
        ],
        out_specs=pl.BlockSpec((MB, _D), lambda i: (i, 0)),
        out_shape=jax.ShapeDtypeStruct((_BSC * N, _D), jnp.float32),
    )(feats, Wpad, b2)

    return jnp.concatenate([out_tc, out_sc.reshape(_BSC, N, _D)], axis=0)


# speedup vs baseline: 1.6919x; 1.0495x over previous
import functools

import jax
import jax.numpy as jnp
from jax import lax
from jax.experimental import pallas as pl
from jax.experimental.pallas import tpu as pltpu
from jax.experimental.pallas import tpu_sc as plsc

_K = 16
_N = 512
_B = 16
_D = 128
_L = 16
_NCHUNK = _N // _L
_NW = 32
_F = 40
_SROWS = 352
_TROWS = _N - _SROWS
_TPB = 2
_RPT = _SROWS // _TPB


def _sc_topk_body(locs_hbm, out_hbm, locv, outbuf):
    cid = lax.axis_index("c")
    sid = lax.axis_index("s")
    wid = sid * 2 + cid
    batch = wid // _TPB
    part = wid % _TPB
    pltpu.sync_copy(locs_hbm.at[batch], locv)

    lane = lax.iota(jnp.int32, _L)
    lane2 = lane * 2
    hrpt = _RPT // 2

    def row_body(r, carry):
        iA = part * _RPT + r
        iB = iA + hrpt
        ivA = jnp.zeros((_L,), jnp.int32) + 2 * iA
        ivB = jnp.zeros((_L,), jnp.int32) + 2 * iB
        xiA = plsc.load_gather(locv, [ivA])
        yiA = plsc.load_gather(locv, [ivA + 1])
        xiB = plsc.load_gather(locv, [ivB])
        yiB = plsc.load_gather(locv, [ivB + 1])
        akA = avA = akB = avB = None
        for c in range(_NCHUNK):
            cols = lane + c * _L
            cols2 = lane2 + c * (2 * _L)
            cx = plsc.load_gather(locv, [cols2])
            cy = plsc.load_gather(locv, [cols2 + 1])
            dxA = cx - xiA
            dyA = cy - yiA
            dxB = cx - xiB
            dyB = cy - yiB
            d2A = dxA * dxA + dyA * dyA
            d2B = dxB * dxB + dyB * dyB
            d2A = jnp.where(cols == iA, jnp.float32(jnp.inf), d2A)
            d2B = jnp.where(cols == iB, jnp.float32(jnp.inf), d2B)
            if c == 0:
                akA, avA = plsc.sort_key_val(d2A, cols)
                akB, avB = plsc.sort_key_val(d2B, cols)
            else:
                bkA, bvA = plsc.sort_key_val(d2A, cols, descending=True)
                bkB, bvB = plsc.sort_key_val(d2B, cols, descending=True)
                tA = bkA < akA
                tB = bkB < akB
                lkA = jnp.where(tA, bkA, akA)
                lvA = jnp.where(tA, bvA, avA)
                lkB = jnp.where(tB, bkB, akB)
                lvB = jnp.where(tB, bvB, avB)
                akA, avA = plsc.sort_key_val(lkA, lvA)
                akB, avB = plsc.sort_key_val(lkB, lvB)
        for (i, xi, yi, av, roff) in ((iA, xiA, yiA, avA, 0),
                                      (iB, xiB, yiB, avB, hrpt)):
            av2 = av * 2
            gx = plsc.load_gather(locv, [av2])
            gy = plsc.load_gather(locv, [av2 + 1])
            head = jnp.where(lane == 1, yi, xi)
            outbuf[r + roff, pl.ds(0, _L)] = head
            rv = jnp.zeros((_L,), jnp.int32) + (r + roff)
            plsc.store_scatter(outbuf, [rv, lane2 + 2], gx - xi)
            plsc.store_scatter(outbuf, [rv, lane2 + 3], gy - yi)
        return carry

    lax.fori_loop(0, hrpt, row_body, 0)
    pltpu.sync_copy(outbuf, out_hbm.at[pl.ds(wid * _RPT, _RPT)])


def _mm_body(feats_ref, W_ref, b_ref, out_ref):
    out_ref[...] = (
        jnp.dot(feats_ref[...], W_ref[...], preferred_element_type=jnp.float32)
        + b_ref[...]
    )


def _tc_body(locsT_ref, locs_ref, Wp_ref, b_ref, out_ref, feats_ref):
    n = _N
    m = _TROWS
    x_row = locsT_ref[0, 0:1, :]
    y_row = locsT_ref[0, 1:2, :]
    x_col = locs_ref[0, :, 0:1]
    y_col = locs_ref[0, :, 1:2]
    dxm = x_row - x_col
    dym = y_row - y_col
    d2 = dxm * dxm + dym * dym
    colj = jax.lax.broadcasted_iota(jnp.int32, (m, n), 1)
    rowi = jax.lax.broadcasted_iota(jnp.int32, (m, n), 0) + _SROWS
    inf = jnp.float32(jnp.inf)
    d2 = jnp.where(rowi == colj, inf, d2)

    feats_ref[:, 0:2] = locs_ref[0]
    feats_ref[:, 34:_F] = jnp.zeros((m, _F - 34), jnp.float32)
    for k in range(_K):
        mind2 = jnp.min(d2, axis=1, keepdims=True)
        mask = d2 == mind2
        feats_ref[:, 2 + k:3 + k] = jnp.sum(
            jnp.where(mask, dxm, 0.0), axis=1, keepdims=True)
        feats_ref[:, 18 + k:19 + k] = jnp.sum(
            jnp.where(mask, dym, 0.0), axis=1, keepdims=True)
        d2 = jnp.where(mask, inf, d2)

    out_ref[0] = (
        jnp.dot(feats_ref[...], Wp_ref[...], preferred_element_type=jnp.float32)
        + b_ref[...]
    )


@jax.jit
def kernel(locs, W, b):
    B, N, _ = locs.shape
    locs_flat = locs.reshape(B, 2 * N)
    b2 = b.reshape(1, _D)

    mesh = plsc.VectorSubcoreMesh(core_axis_name="c", subcore_axis_name="s")
    feats = pl.kernel(
        _sc_topk_body,
        out_type=jax.ShapeDtypeStruct((_B * _SROWS, _F), jnp.float32),
        mesh=mesh,
        compiler_params=pltpu.CompilerParams(needs_layout_passes=False),
        scratch_types=[
            pltpu.VMEM((2 * N,), jnp.float32),
            pltpu.VMEM((_RPT, _F), jnp.float32),
        ],
    )(locs_flat)

    locsT = locs.transpose(0, 2, 1)
    locs_tc = locs[:, _SROWS:, :]
    order = [0, 1] + [2 + 2 * k for k in range(_K)] + [3 + 2 * k for k in range(_K)]
    Wp = jnp.zeros((_F, _D), W.dtype).at[:34].set(W[jnp.asarray(order)])
    out_tc = pl.pallas_call(
        _tc_body,
        grid=(_B,),
        in_specs=[
            pl.BlockSpec((1, 2, N), lambda i: (i, 0, 0)),
            pl.BlockSpec((1, _TROWS, 2), lambda i: (i, 0, 0)),
            pl.BlockSpec((_F, _D), lambda i: (0, 0)),
            pl.BlockSpec((1, _D), lambda i: (0, 0)),
        ],
        out_specs=pl.BlockSpec((1, _TROWS, _D), lambda i: (i, 0, 0)),
        out_shape=jax.ShapeDtypeStruct((_B, _TROWS, _D), jnp.float32),
        scratch_shapes=[pltpu.VMEM((_TROWS, _F), jnp.float32)],
    )(locsT, locs_tc, Wp, b2)

    Wpad = jnp.zeros((_F, _D), W.dtype).at[:34].set(W)
    MB = 512
    out_sc = pl.pallas_call(
        _mm_body,
        grid=(_B * _SROWS // MB,),
        in_specs=[
            pl.BlockSpec((MB, _F), lambda i: (i, 0)),
            pl.BlockSpec((_F, _D), lambda i: (0, 0)),
            pl.BlockSpec((1, _D), lambda i: (0, 0)),
        ],
        out_specs=pl.BlockSpec((MB, _D), lambda i: (i, 0)),
        out_shape=jax.ShapeDtypeStruct((_B * _SROWS, _D), jnp.float32),
    )(feats, Wpad, b2)

    return jnp.concatenate(
        [out_sc.reshape(_B, _SROWS, _D), out_tc], axis=1)


# speedup vs baseline: 1.7146x; 1.0134x over previous
import functools

import jax
import jax.numpy as jnp
from jax import lax
from jax.experimental import pallas as pl
from jax.experimental.pallas import tpu as pltpu
from jax.experimental.pallas import tpu_sc as plsc

_K = 16
_N = 512
_B = 16
_D = 128
_L = 16
_NCHUNK = _N // _L
_NW = 32
_F = 40
_SROWS = 352
_TROWS = _N - _SROWS
_TPB = 2
_RPT = _SROWS // _TPB


def _sc_topk_body(locs_hbm, out_hbm, locv, outbuf):
    cid = lax.axis_index("c")
    sid = lax.axis_index("s")
    wid = sid * 2 + cid
    batch = wid // _TPB
    part = wid % _TPB
    pltpu.sync_copy(locs_hbm.at[batch], locv)

    lane = lax.iota(jnp.int32, _L)
    lane2 = lane * 2
    hrpt = _RPT // 2

    def row_body(r, carry):
        iA = part * _RPT + r
        iB = iA + hrpt
        ivA = jnp.zeros((_L,), jnp.int32) + 2 * iA
        ivB = jnp.zeros((_L,), jnp.int32) + 2 * iB
        xiA = plsc.load_gather(locv, [ivA])
        yiA = plsc.load_gather(locv, [ivA + 1])
        xiB = plsc.load_gather(locv, [ivB])
        yiB = plsc.load_gather(locv, [ivB + 1])
        akA = avA = akB = avB = None
        for c in range(_NCHUNK):
            cols = lane + c * _L
            cols2 = lane2 + c * (2 * _L)
            cx = plsc.load_gather(locv, [cols2])
            cy = plsc.load_gather(locv, [cols2 + 1])
            dxA = cx - xiA
            dyA = cy - yiA
            dxB = cx - xiB
            dyB = cy - yiB
            d2A = dxA * dxA + dyA * dyA
            d2B = dxB * dxB + dyB * dyB
            d2A = jnp.where(cols == iA, jnp.float32(jnp.inf), d2A)
            d2B = jnp.where(cols == iB, jnp.float32(jnp.inf), d2B)
            if c == 0:
                akA, avA = plsc.sort_key_val(d2A, cols)
                akB, avB = plsc.sort_key_val(d2B, cols)
            else:
                bkA, bvA = plsc.sort_key_val(d2A, cols, descending=True)
                bkB, bvB = plsc.sort_key_val(d2B, cols, descending=True)
                tA = bkA < akA
                tB = bkB < akB
                lkA = jnp.where(tA, bkA, akA)
                lvA = jnp.where(tA, bvA, avA)
                lkB = jnp.where(tB, bkB, akB)
                lvB = jnp.where(tB, bvB, avB)
                akA, avA = plsc.sort_key_val(lkA, lvA)
                akB, avB = plsc.sort_key_val(lkB, lvB)
        for (i, xi, yi, av, roff) in ((iA, xiA, yiA, avA, 0),
                                      (iB, xiB, yiB, avB, hrpt)):
            av2 = av * 2
            gx = plsc.load_gather(locv, [av2])
            gy = plsc.load_gather(locv, [av2 + 1])
            head = jnp.where(lane == 1, yi, xi)
            outbuf[r + roff, pl.ds(0, _L)] = head
            rv = jnp.zeros((_L,), jnp.int32) + (r + roff)
            plsc.store_scatter(outbuf, [rv, lane2 + 2], gx - xi)
            plsc.store_scatter(outbuf, [rv, lane2 + 3], gy - yi)
        return carry

    lax.fori_loop(0, hrpt, row_body, 0)
    pltpu.sync_copy(outbuf, out_hbm.at[pl.ds(wid * _RPT, _RPT)])


def _mm_body(feats_ref, tc_ref, W_ref, b_ref, out_ref):
    out_ref[0, 0:_SROWS, :] = (
        jnp.dot(feats_ref[0], W_ref[...], preferred_element_type=jnp.float32)
        + b_ref[...]
    )
    out_ref[0, _SROWS:_N, :] = tc_ref[0]


def _tc_body(locsT_ref, locs_ref, Wp_ref, b_ref, out_ref, feats_ref):
    n = _N
    m = _TROWS
    x_row = locsT_ref[0, 0:1, :]
    y_row = locsT_ref[0, 1:2, :]
    x_col = locs_ref[0, :, 0:1]
    y_col = locs_ref[0, :, 1:2]
    dxm = x_row - x_col
    dym = y_row - y_col
    d2 = dxm * dxm + dym * dym
    colj = jax.lax.broadcasted_iota(jnp.int32, (m, n), 1)
    rowi = jax.lax.broadcasted_iota(jnp.int32, (m, n), 0) + _SROWS
    inf = jnp.float32(jnp.inf)
    d2 = jnp.where(rowi == colj, inf, d2)

    feats_ref[:, 0:2] = locs_ref[0]
    feats_ref[:, 34:_F] = jnp.zeros((m, _F - 34), jnp.float32)
    for k in range(_K):
        mind2 = jnp.min(d2, axis=1, keepdims=True)
        mask = d2 == mind2
        feats_ref[:, 2 + k:3 + k] = jnp.sum(
            jnp.where(mask, dxm, 0.0), axis=1, keepdims=True)
        feats_ref[:, 18 + k:19 + k] = jnp.sum(
            jnp.where(mask, dym, 0.0), axis=1, keepdims=True)
        d2 = jnp.where(mask, inf, d2)

    out_ref[0] = (
        jnp.dot(feats_ref[...], Wp_ref[...], preferred_element_type=jnp.float32)
        + b_ref[...]
    )


@jax.jit
def kernel(locs, W, b):
    B, N, _ = locs.shape
    locs_flat = locs.reshape(B, 2 * N)
    b2 = b.reshape(1, _D)

    mesh = plsc.VectorSubcoreMesh(core_axis_name="c", subcore_axis_name="s")
    feats = pl.kernel(
        _sc_topk_body,
        out_type=jax.ShapeDtypeStruct((_B * _SROWS, _F), jnp.float32),
        mesh=mesh,
        compiler_params=pltpu.CompilerParams(needs_layout_passes=False),
        scratch_types=[
            pltpu.VMEM((2 * N,), jnp.float32),
            pltpu.VMEM((_RPT, _F), jnp.float32),
        ],
    )(locs_flat)

    locsT = locs.transpose(0, 2, 1)
    locs_tc = locs[:, _SROWS:, :]
    order = [0, 1] + [2 + 2 * k for k in range(_K)] + [3 + 2 * k for k in range(_K)]
    Wp = jnp.zeros((_F, _D), W.dtype).at[:34].set(W[jnp.asarray(order)])
    out_tc = pl.pallas_call(
        _tc_body,
        grid=(_B,),
        in_specs=[
            pl.BlockSpec((1, 2, N), lambda i: (i, 0, 0)),
            pl.BlockSpec((1, _TROWS, 2), lambda i: (i, 0, 0)),
            pl.BlockSpec((_F, _D), lambda i: (0, 0)),
            pl.BlockSpec((1, _D), lambda i: (0, 0)),
        ],
        out_specs=pl.BlockSpec((1, _TROWS, _D), lambda i: (i, 0, 0)),
        out_shape=jax.ShapeDtypeStruct((_B, _TROWS, _D), jnp.float32),
        scratch_shapes=[pltpu.VMEM((_TROWS, _F), jnp.float32)],
    )(locsT, locs_tc, Wp, b2)

    Wpad = jnp.zeros((_F, _D), W.dtype).at[:34].set(W)
    feats3 = feats.reshape(_B, _SROWS, _F)
    out = pl.pallas_call(
        _mm_body,
        grid=(_B,),
        in_specs=[
            pl.BlockSpec((1, _SROWS, _F), lambda i: (i, 0, 0)),
            pl.BlockSpec((1, _TROWS, _D), lambda i: (i, 0, 0)),
            pl.BlockSpec((_F, _D), lambda i: (0, 0)),
            pl.BlockSpec((1, _D), lambda i: (0, 0)),
        ],
        out_specs=pl.BlockSpec((1, _N, _D), lambda i: (i, 0, 0)),
        out_shape=jax.ShapeDtypeStruct((_B, _N, _D), jnp.float32),
    )(feats3, out_tc, Wpad, b2)
    return out


# speedup vs baseline: 1.8462x; 1.0767x over previous
import functools

import jax
import jax.numpy as jnp
from jax import lax
from jax.experimental import pallas as pl
from jax.experimental.pallas import tpu as pltpu
from jax.experimental.pallas import tpu_sc as plsc

_K = 16
_N = 512
_B = 16
_D = 128
_L = 16
_NCHUNK = _N // _L
_NW = 32
_F = 40
_SROWS = 384
_TROWS = _N - _SROWS
_TPB = 2
_RPT = _SROWS // _TPB


def _sc_topk_body(locs_hbm, out_hbm, locv, outbuf):
    cid = lax.axis_index("c")
    sid = lax.axis_index("s")
    wid = sid * 2 + cid
    batch = wid // _TPB
    part = wid % _TPB
    pltpu.sync_copy(locs_hbm.at[batch], locv)

    lane = lax.iota(jnp.int32, _L)
    lane2 = lane * 2
    nway = 3
    grp = _RPT // nway

    def row_body(r, carry):
        idxs = [part * _RPT + r + j * grp for j in range(nway)]
        xis, yis = [], []
        for i in idxs:
            iv = jnp.zeros((_L,), jnp.int32) + 2 * i
            xis.append(plsc.load_gather(locv, [iv]))
            yis.append(plsc.load_gather(locv, [iv + 1]))
        aks = [None] * nway
        avs = [None] * nway
        for c in range(_NCHUNK):
            cols = lane + c * _L
            cols2 = lane2 + c * (2 * _L)
            cx = plsc.load_gather(locv, [cols2])
            cy = plsc.load_gather(locv, [cols2 + 1])
            d2s = []
            for j in range(nway):
                dx = cx - xis[j]
                dy = cy - yis[j]
                d2 = dx * dx + dy * dy
                d2s.append(jnp.where(cols == idxs[j], jnp.float32(jnp.inf), d2))
            if c == 0:
                for j in range(nway):
                    aks[j], avs[j] = plsc.sort_key_val(d2s[j], cols)
            else:
                bs = [plsc.sort_key_val(d2s[j], cols, descending=True)
                      for j in range(nway)]
                for j in range(nway):
                    bk, bv = bs[j]
                    t = bk < aks[j]
                    lk = jnp.where(t, bk, aks[j])
                    lv = jnp.where(t, bv, avs[j])
                    aks[j], avs[j] = plsc.sort_key_val(lk, lv)
        for j in range(nway):
            av2 = avs[j] * 2
            gx = plsc.load_gather(locv, [av2])
            gy = plsc.load_gather(locv, [av2 + 1])
            head = jnp.where(lane == 1, yis[j], xis[j])
            ro = r + j * grp
            outbuf[ro, pl.ds(0, _L)] = head
            rv = jnp.zeros((_L,), jnp.int32) + ro
            plsc.store_scatter(outbuf, [rv, lane2 + 2], gx - xis[j])
            plsc.store_scatter(outbuf, [rv, lane2 + 3], gy - yis[j])
        return carry

    lax.fori_loop(0, grp, row_body, 0)
    pltpu.sync_copy(outbuf, out_hbm.at[pl.ds(wid * _RPT, _RPT)])


def _mm_body(feats_ref, tc_ref, W_ref, b_ref, out_ref):
    out_ref[0, 0:_SROWS, :] = (
        jnp.dot(feats_ref[0], W_ref[...], preferred_element_type=jnp.float32)
        + b_ref[...]
    )
    out_ref[0, _SROWS:_N, :] = tc_ref[0]


def _tc_body(locsT_ref, locs_ref, Wp_ref, b_ref, out_ref, feats_ref):
    n = _N
    m = _TROWS
    x_row = locsT_ref[0, 0:1, :]
    y_row = locsT_ref[0, 1:2, :]
    x_col = locs_ref[0, :, 0:1]
    y_col = locs_ref[0, :, 1:2]
    dxm = x_row - x_col
    dym = y_row - y_col
    d2 = dxm * dxm + dym * dym
    colj = jax.lax.broadcasted_iota(jnp.int32, (m, n), 1)
    rowi = jax.lax.broadcasted_iota(jnp.int32, (m, n), 0) + _SROWS
    inf = jnp.float32(jnp.inf)
    d2 = jnp.where(rowi == colj, inf, d2)

    feats_ref[:, 0:2] = locs_ref[0]
    feats_ref[:, 34:_F] = jnp.zeros((m, _F - 34), jnp.float32)
    for k in range(_K):
        mind2 = jnp.min(d2, axis=1, keepdims=True)
        mask = d2 == mind2
        feats_ref[:, 2 + k:3 + k] = jnp.sum(
            jnp.where(mask, dxm, 0.0), axis=1, keepdims=True)
        feats_ref[:, 18 + k:19 + k] = jnp.sum(
            jnp.where(mask, dym, 0.0), axis=1, keepdims=True)
        d2 = jnp.where(mask, inf, d2)

    out_ref[0] = (
        jnp.dot(feats_ref[...], Wp_ref[...], preferred_element_type=jnp.float32)
        + b_ref[...]
    )


@jax.jit
def kernel(locs, W, b):
    B, N, _ = locs.shape
    locs_flat = locs.reshape(B, 2 * N)
    b2 = b.reshape(1, _D)

    mesh = plsc.VectorSubcoreMesh(core_axis_name="c", subcore_axis_name="s")
    feats = pl.kernel(
        _sc_topk_body,
        out_type=jax.ShapeDtypeStruct((_B * _SROWS, _F), jnp.float32),
        mesh=mesh,
        compiler_params=pltpu.CompilerParams(needs_layout_passes=False),
        scratch_types=[
            pltpu.VMEM((2 * N,), jnp.float32),
            pltpu.VMEM((_RPT, _F), jnp.float32),
        ],
    )(locs_flat)

    locsT = locs.transpose(0, 2, 1)
    locs_tc = locs[:, _SROWS:, :]
    order = [0, 1] + [2 + 2 * k for k in range(_K)] + [3 + 2 * k for k in range(_K)]
    Wp = jnp.zeros((_F, _D), W.dtype).at[:34].set(W[jnp.asarray(order)])
    out_tc = pl.pallas_call(
        _tc_body,
        grid=(_B,),
        in_specs=[
            pl.BlockSpec((1, 2, N), lambda i: (i, 0, 0)),
            pl.BlockSpec((1, _TROWS, 2), lambda i: (i, 0, 0)),
            pl.BlockSpec((_F, _D), lambda i: (0, 0)),
            pl.BlockSpec((1, _D), lambda i: (0, 0)),
        ],
        out_specs=pl.BlockSpec((1, _TROWS, _D), lambda i: (i, 0, 0)),
        out_shape=jax.ShapeDtypeStruct((_B, _TROWS, _D), jnp.float32),
        scratch_shapes=[pltpu.VMEM((_TROWS, _F), jnp.float32)],
    )(locsT, locs_tc, Wp, b2)

    Wpad = jnp.zeros((_F, _D), W.dtype).at[:34].set(W)
    feats3 = feats.reshape(_B, _SROWS, _F)
    out = pl.pallas_call(
        _mm_body,
        grid=(_B,),
        in_specs=[
            pl.BlockSpec((1, _SROWS, _F), lambda i: (i, 0, 0)),
            pl.BlockSpec((1, _TROWS, _D), lambda i: (i, 0, 0)),
            pl.BlockSpec((_F, _D), lambda i: (0, 0)),
            pl.BlockSpec((1, _D), lambda i: (0, 0)),
        ],
        out_specs=pl.BlockSpec((1, _N, _D), lambda i: (i, 0, 0)),
        out_shape=jax.ShapeDtypeStruct((_B, _N, _D), jnp.float32),
    )(feats3, out_tc, Wpad, b2)
    return out


# speedup vs baseline: 2.0409x; 1.1055x over previous
import functools

import jax
import jax.numpy as jnp
from jax import lax
from jax.experimental import pallas as pl
from jax.experimental.pallas import tpu as pltpu
from jax.experimental.pallas import tpu_sc as plsc

_K = 16
_N = 512
_B = 16
_D = 128
_L = 16
_NCHUNK = _N // _L
_NW = 32
_F = 40
_SROWS = 384
_TROWS = _N - _SROWS
_TPB = 2
_RPT = _SROWS // _TPB


def _sc_topk_body(locs_hbm, out_hbm, locv, outbuf):
    cid = lax.axis_index("c")
    sid = lax.axis_index("s")
    wid = sid * 2 + cid
    batch = wid // _TPB
    part = wid % _TPB
    pltpu.sync_copy(locs_hbm.at[batch], locv)

    lane = lax.iota(jnp.int32, _L)
    lane2 = lane * 2
    nway = 3
    grp = _RPT // nway

    def row_body(r, carry):
        idxs = [part * _RPT + r + j * grp for j in range(nway)]
        xis, yis = [], []
        for i in idxs:
            iv = jnp.zeros((_L,), jnp.int32) + 2 * i
            xis.append(plsc.load_gather(locv, [iv]))
            yis.append(plsc.load_gather(locv, [iv + 1]))
        aks = [None] * nway
        avs = [None] * nway
        for c in range(_NCHUNK):
            cols = lane + c * _L
            cols2 = lane2 + c * (2 * _L)
            cx = plsc.load_gather(locv, [cols2])
            cy = plsc.load_gather(locv, [cols2 + 1])
            d2s = []
            for j in range(nway):
                dx = cx - xis[j]
                dy = cy - yis[j]
                d2 = dx * dx + dy * dy
                d2s.append(jnp.where(cols == idxs[j], jnp.float32(jnp.inf), d2))
            if c == 0:
                for j in range(nway):
                    aks[j], avs[j] = plsc.sort_key_val(d2s[j], cols)
            else:
                bs = [plsc.sort_key_val(d2s[j], cols, descending=True)
                      for j in range(nway)]
                for j in range(nway):
                    bk, bv = bs[j]
                    t = bk < aks[j]
                    lk = jnp.where(t, bk, aks[j])
                    lv = jnp.where(t, bv, avs[j])
                    aks[j], avs[j] = plsc.sort_key_val(lk, lv)
        for j in range(nway):
            av2 = avs[j] * 2
            gx = plsc.load_gather(locv, [av2])
            gy = plsc.load_gather(locv, [av2 + 1])
            head = jnp.where(lane == 1, yis[j], xis[j])
            ro = r + j * grp
            outbuf[ro, pl.ds(0, _L)] = head
            rv = jnp.zeros((_L,), jnp.int32) + ro
            plsc.store_scatter(outbuf, [rv, lane2 + 2], gx - xis[j])
            plsc.store_scatter(outbuf, [rv, lane2 + 3], gy - yis[j])
        return carry

    lax.fori_loop(0, grp, row_body, 0)
    pltpu.sync_copy(outbuf, out_hbm.at[pl.ds(wid * _RPT, _RPT)])


_MMG = 4


def _mm_body(feats_ref, tc_ref, W_ref, b_ref, out_ref):
    for j in range(_MMG):
        out_ref[j, 0:_SROWS, :] = (
            jnp.dot(feats_ref[j], W_ref[...],
                    preferred_element_type=jnp.float32)
            + b_ref[...]
        )
        out_ref[j, _SROWS:_N, :] = tc_ref[j]


def _tc_body(locsT_ref, locs_ref, Wp_ref, b_ref, out_ref, feats_ref):
    n = _N
    m = _TROWS
    x_row = locsT_ref[0, 0:1, :]
    y_row = locsT_ref[0, 1:2, :]
    x_col = locs_ref[0, :, 0:1]
    y_col = locs_ref[0, :, 1:2]
    dxm = x_row - x_col
    dym = y_row - y_col
    d2 = dxm * dxm + dym * dym
    colj = jax.lax.broadcasted_iota(jnp.int32, (m, n), 1)
    rowi = jax.lax.broadcasted_iota(jnp.int32, (m, n), 0) + _SROWS
    inf = jnp.float32(jnp.inf)
    d2 = jnp.where(rowi == colj, inf, d2)

    feats_ref[:, 0:2] = locs_ref[0]
    feats_ref[:, 34:_F] = jnp.zeros((m, _F - 34), jnp.float32)
    for k in range(_K):
        mind2 = jnp.min(d2, axis=1, keepdims=True)
        mask = d2 == mind2
        feats_ref[:, 2 + k:3 + k] = jnp.sum(
            jnp.where(mask, dxm, 0.0), axis=1, keepdims=True)
        feats_ref[:, 18 + k:19 + k] = jnp.sum(
            jnp.where(mask, dym, 0.0), axis=1, keepdims=True)
        d2 = jnp.where(mask, inf, d2)

    out_ref[0] = (
        jnp.dot(feats_ref[...], Wp_ref[...], preferred_element_type=jnp.float32)
        + b_ref[...]
    )


@jax.jit
def kernel(locs, W, b):
    B, N, _ = locs.shape
    locs_flat = locs.reshape(B, 2 * N)
    b2 = b.reshape(1, _D)

    mesh = plsc.VectorSubcoreMesh(core_axis_name="c", subcore_axis_name="s")
    feats = pl.kernel(
        _sc_topk_body,
        out_type=jax.ShapeDtypeStruct((_B * _SROWS, _F), jnp.float32),
        mesh=mesh,
        compiler_params=pltpu.CompilerParams(needs_layout_passes=False),
        scratch_types=[
            pltpu.VMEM((2 * N,), jnp.float32),
            pltpu.VMEM((_RPT, _F), jnp.float32),
        ],
    )(locs_flat)

    locsT = locs.transpose(0, 2, 1)
    locs_tc = locs[:, _SROWS:, :]
    order = [0, 1] + [2 + 2 * k for k in range(_K)] + [3 + 2 * k for k in range(_K)]
    Wp = jnp.zeros((_F, _D), W.dtype).at[:34].set(W[jnp.asarray(order)])
    out_tc = pl.pallas_call(
        _tc_body,
        grid=(_B,),
        in_specs=[
            pl.BlockSpec((1, 2, N), lambda i: (i, 0, 0)),
            pl.BlockSpec((1, _TROWS, 2), lambda i: (i, 0, 0)),
            pl.BlockSpec((_F, _D), lambda i: (0, 0)),
            pl.BlockSpec((1, _D), lambda i: (0, 0)),
        ],
        out_specs=pl.BlockSpec((1, _TROWS, _D), lambda i: (i, 0, 0)),
        out_shape=jax.ShapeDtypeStruct((_B, _TROWS, _D), jnp.float32),
        scratch_shapes=[pltpu.VMEM((_TROWS, _F), jnp.float32)],
    )(locsT, locs_tc, Wp, b2)

    Wpad = jnp.zeros((_F, _D), W.dtype).at[:34].set(W)
    feats3 = feats.reshape(_B, _SROWS, _F)
    out = pl.pallas_call(
        _mm_body,
        grid=(_B // _MMG,),
        in_specs=[
            pl.BlockSpec((_MMG, _SROWS, _F), lambda i: (i, 0, 0)),
            pl.BlockSpec((_MMG, _TROWS, _D), lambda i: (i, 0, 0)),
            pl.BlockSpec((_F, _D), lambda i: (0, 0)),
            pl.BlockSpec((1, _D), lambda i: (0, 0)),
        ],
        out_specs=pl.BlockSpec((_MMG, _N, _D), lambda i: (i, 0, 0)),
        out_shape=jax.ShapeDtypeStruct((_B, _N, _D), jnp.float32),
    )(feats3, out_tc, Wpad, b2)
    return out


# speedup vs baseline: 2.1452x; 1.0511x over previous
import functools

import jax
import jax.numpy as jnp
from jax import lax
from jax.experimental import pallas as pl
from jax.experimental.pallas import tpu as pltpu
from jax.experimental.pallas import tpu_sc as plsc

_K = 16
_N = 512
_B = 16
_D = 128
_L = 16
_NCHUNK = _N // _L
_NW = 32
_F = 40
_SROWS = 384
_TROWS = _N - _SROWS
_TPB = 2
_RPT = _SROWS // _TPB


def _sc_topk_body(locs_hbm, out_hbm, locv, outbuf):
    cid = lax.axis_index("c")
    sid = lax.axis_index("s")
    wid = sid * 2 + cid
    batch = wid // _TPB
    part = wid % _TPB
    pltpu.sync_copy(locs_hbm.at[batch], locv)

    lane = lax.iota(jnp.int32, _L)
    lane2 = lane * 2
    nway = 4
    grp = _RPT // nway

    def row_body(r, carry):
        idxs = [part * _RPT + r + j * grp for j in range(nway)]
        xis, yis = [], []
        for i in idxs:
            iv = jnp.zeros((_L,), jnp.int32) + 2 * i
            xis.append(plsc.load_gather(locv, [iv]))
            yis.append(plsc.load_gather(locv, [iv + 1]))
        aks = [None] * nway
        avs = [None] * nway
        for c in range(_NCHUNK):
            cols = lane + c * _L
            cols2 = lane2 + c * (2 * _L)
            cx = plsc.load_gather(locv, [cols2])
            cy = plsc.load_gather(locv, [cols2 + 1])
            d2s = []
            for j in range(nway):
                dx = cx - xis[j]
                dy = cy - yis[j]
                d2 = dx * dx + dy * dy
                d2s.append(jnp.where(cols == idxs[j], jnp.float32(jnp.inf), d2))
            if c == 0:
                for j in range(nway):
                    aks[j], avs[j] = plsc.sort_key_val(d2s[j], cols)
            else:
                bs = [plsc.sort_key_val(d2s[j], cols, descending=True)
                      for j in range(nway)]
                for j in range(nway):
                    bk, bv = bs[j]
                    t = bk < aks[j]
                    lk = jnp.where(t, bk, aks[j])
                    lv = jnp.where(t, bv, avs[j])
                    aks[j], avs[j] = plsc.sort_key_val(lk, lv)
        for j in range(nway):
            av2 = avs[j] * 2
            gx = plsc.load_gather(locv, [av2])
            gy = plsc.load_gather(locv, [av2 + 1])
            head = jnp.where(lane == 1, yis[j], xis[j])
            ro = r + j * grp
            outbuf[ro, pl.ds(0, _L)] = head
            rv = jnp.zeros((_L,), jnp.int32) + ro
            plsc.store_scatter(outbuf, [rv, lane2 + 2], gx - xis[j])
            plsc.store_scatter(outbuf, [rv, lane2 + 3], gy - yis[j])
        return carry

    lax.fori_loop(0, grp, row_body, 0)
    pltpu.sync_copy(outbuf, out_hbm.at[pl.ds(wid * _RPT, _RPT)])


_MMG = 4


def _mm_body(feats_ref, tc_ref, W_ref, b_ref, out_ref):
    for j in range(_MMG):
        out_ref[j, 0:_SROWS, :] = (
            jnp.dot(feats_ref[j], W_ref[...],
                    preferred_element_type=jnp.float32)
            + b_ref[...]
        )
        out_ref[j, _SROWS:_N, :] = tc_ref[j]


def _tc_body(locsT_ref, locs_ref, Wp_ref, b_ref, out_ref, feats_ref):
    n = _N
    m = _TROWS
    x_row = locsT_ref[0, 0:1, :]
    y_row = locsT_ref[0, 1:2, :]
    x_col = locs_ref[0, :, 0:1]
    y_col = locs_ref[0, :, 1:2]
    dxm = x_row - x_col
    dym = y_row - y_col
    d2 = dxm * dxm + dym * dym
    colj = jax.lax.broadcasted_iota(jnp.int32, (m, n), 1)
    rowi = jax.lax.broadcasted_iota(jnp.int32, (m, n), 0) + _SROWS
    inf = jnp.float32(jnp.inf)
    d2 = jnp.where(rowi == colj, inf, d2)

    feats_ref[:, 0:2] = locs_ref[0]
    feats_ref[:, 34:_F] = jnp.zeros((m, _F - 34), jnp.float32)
    for k in range(_K):
        mind2 = jnp.min(d2, axis=1, keepdims=True)
        mask = d2 == mind2
        feats_ref[:, 2 + k:3 + k] = jnp.sum(
            jnp.where(mask, dxm, 0.0), axis=1, keepdims=True)
        feats_ref[:, 18 + k:19 + k] = jnp.sum(
            jnp.where(mask, dym, 0.0), axis=1, keepdims=True)
        d2 = jnp.where(mask, inf, d2)

    out_ref[0] = (
        jnp.dot(feats_ref[...], Wp_ref[...], preferred_element_type=jnp.float32)
        + b_ref[...]
    )


@jax.jit
def kernel(locs, W, b):
    B, N, _ = locs.shape
    locs_flat = locs.reshape(B, 2 * N)
    b2 = b.reshape(1, _D)

    mesh = plsc.VectorSubcoreMesh(core_axis_name="c", subcore_axis_name="s")
    feats = pl.kernel(
        _sc_topk_body,
        out_type=jax.ShapeDtypeStruct((_B * _SROWS, _F), jnp.float32),
        mesh=mesh,
        compiler_params=pltpu.CompilerParams(needs_layout_passes=False),
        scratch_types=[
            pltpu.VMEM((2 * N,), jnp.float32),
            pltpu.VMEM((_RPT, _F), jnp.float32),
        ],
    )(locs_flat)

    locsT = locs.transpose(0, 2, 1)
    locs_tc = locs[:, _SROWS:, :]
    order = [0, 1] + [2 + 2 * k for k in range(_K)] + [3 + 2 * k for k in range(_K)]
    Wp = jnp.zeros((_F, _D), W.dtype).at[:34].set(W[jnp.asarray(order)])
    out_tc = pl.pallas_call(
        _tc_body,
        grid=(_B,),
        in_specs=[
            pl.BlockSpec((1, 2, N), lambda i: (i, 0, 0)),
            pl.BlockSpec((1, _TROWS, 2), lambda i: (i, 0, 0)),
            pl.BlockSpec((_F, _D), lambda i: (0, 0)),
            pl.BlockSpec((1, _D), lambda i: (0, 0)),
        ],
        out_specs=pl.BlockSpec((1, _TROWS, _D), lambda i: (i, 0, 0)),
        out_shape=jax.ShapeDtypeStruct((_B, _TROWS, _D), jnp.float32),
        scratch_shapes=[pltpu.VMEM((_TROWS, _F), jnp.float32)],
    )(locsT, locs_tc, Wp, b2)

    Wpad = jnp.zeros((_F, _D), W.dtype).at[:34].set(W)
    feats3 = feats.reshape(_B, _SROWS, _F)
    out = pl.pallas_call(
        _mm_body,
        grid=(_B // _MMG,),
        in_specs=[
            pl.BlockSpec((_MMG, _SROWS, _F), lambda i: (i, 0, 0)),
            pl.BlockSpec((_MMG, _TROWS, _D), lambda i: (i, 0, 0)),
            pl.BlockSpec((_F, _D), lambda i: (0, 0)),
            pl.BlockSpec((1, _D), lambda i: (0, 0)),
        ],
        out_specs=pl.BlockSpec((_MMG, _N, _D), lambda i: (i, 0, 0)),
        out_shape=jax.ShapeDtypeStruct((_B, _N, _D), jnp.float32),
    )(feats3, out_tc, Wpad, b2)
    return out
